# Initial kernel scaffold; baseline (speedup 1.0000x reference)
import jax
import jax.numpy as jnp
from jax.experimental import pallas as pl


def kernel(data, edge_index, W_bases, W_comb, b_comb, conv_bias, bn_gamma, bn_beta, fc_W, fc_b):
    raise NotImplementedError("write your pallas kernel here")


# speedup vs baseline: 2.4607x; 2.4607x over previous
import functools
import jax
import jax.numpy as jnp
from jax.experimental import pallas as pl
from jax.experimental.pallas import tpu as pltpu

N = 10000
E = 320000
C = 128
H = 8
NB = 4
A = 4
FH = C // H
L = 3
D = NB * FH * L
DW = H * NB * A * L
R = 400
G = N // R


def _proj_body(x_ref, wb_ref, wc_ref, bc_ref, bases_ref, w_ref):
    x = x_ref[...]
    bases_ref[...] = jnp.dot(x, wb_ref[...], preferred_element_type=jnp.float32)
    w_ref[...] = jnp.dot(x, wc_ref[...], preferred_element_type=jnp.float32) + bc_ref[...][0:1, :]


def _proj(data, Wb_all, Wc_all, bc_all):
    return pl.pallas_call(
        _proj_body,
        grid=(G,),
        in_specs=[
            pl.BlockSpec((R, C), lambda i: (i, 0)),
            pl.BlockSpec((C, D), lambda i: (0, 0)),
            pl.BlockSpec((C, DW), lambda i: (0, 0)),
            pl.BlockSpec((8, DW), lambda i: (0, 0)),
        ],
        out_specs=[
            pl.BlockSpec((R, D), lambda i: (i, 0)),
            pl.BlockSpec((R, DW), lambda i: (i, 0)),
        ],
        out_shape=[
            jax.ShapeDtypeStruct((N, D), jnp.float32),
            jax.ShapeDtypeStruct((N, DW), jnp.float32),
        ],
    )(data, Wb_all, Wc_all, bc_all)


def _combine_body(bases_ref, w_ref, sum_ref, max_ref, sym_ref, rdeg_ref, dis_ref,
                  conv_ref, stats_ref):
    step = pl.program_id(0)
    bases = bases_ref[...]
    rdeg = rdeg_ref[...]
    dis = dis_ref[...]
    dis2 = dis * dis
    outs = []
    for l in range(L):
        sl = slice(l * 64, (l + 1) * 64)
        b_l = bases[:, sl]
        s_full = sum_ref[...][:, sl] + b_l
        mean = s_full * rdeg
        mx = jnp.maximum(max_ref[...][:, sl], b_l)
        sym = dis * sym_ref[...][:, sl] + dis2 * b_l
        aggs = (mean, mx, s_full, sym)
        w3 = w_ref[...][:, l * 128:(l + 1) * 128].reshape(R, H, 16)
        acc = jnp.zeros((R, C), jnp.float32)
        for j in range(16):
            a_, b_ = j // NB, j % NB
            aj = aggs[a_][:, b_ * FH:(b_ + 1) * FH]
            ajt = jnp.concatenate([aj] * H, axis=1)
            wj = jnp.broadcast_to(w3[:, :, j:j + 1], (R, H, FH)).reshape(R, C)
            acc = acc + wj * ajt
        outs.append(acc)
    h_all = jnp.concatenate(outs, axis=1)
    conv_ref[...] = h_all
    psum = jnp.sum(h_all, axis=0, keepdims=True)
    psumsq = jnp.sum(h_all * h_all, axis=0, keepdims=True)
    blk = jnp.concatenate([psum, psumsq, jnp.zeros((6, DW), jnp.float32)], axis=0)

    @pl.when(step == 0)
    def _():
        stats_ref[...] = jnp.zeros((8, DW), jnp.float32)

    stats_ref[...] += blk


def _combine(bases, w_all, sum_e, max_e, sym_e, rdeg, dis):
    return pl.pallas_call(
        _combine_body,
        grid=(G,),
        in_specs=[
            pl.BlockSpec((R, D), lambda i: (i, 0)),
            pl.BlockSpec((R, DW), lambda i: (i, 0)),
            pl.BlockSpec((R, D), lambda i: (i, 0)),
            pl.BlockSpec((R, D), lambda i: (i, 0)),
            pl.BlockSpec((R, D), lambda i: (i, 0)),
            pl.BlockSpec((R, 1), lambda i: (i, 0)),
            pl.BlockSpec((R, 1), lambda i: (i, 0)),
        ],
        out_specs=[
            pl.BlockSpec((R, DW), lambda i: (i, 0)),
            pl.BlockSpec((8, DW), lambda i: (0, 0)),
        ],
        out_shape=[
            jax.ShapeDtypeStruct((N, DW), jnp.float32),
            jax.ShapeDtypeStruct((8, DW), jnp.float32),
        ],
    )(bases, w_all, sum_e, max_e, sym_e, rdeg, dis)


def _final_body(data_ref, conv_ref, scale_ref, shift_ref, fcw_ref, out_ref):
    acc = jnp.dot(data_ref[...], fcw_ref[...][0:C, :],
                  preferred_element_type=jnp.float32)
    conv = conv_ref[...]
    scale = scale_ref[...][0:1, :]
    shift = shift_ref[...][0:1, :]
    for l in range(L):
        h = conv[:, l * C:(l + 1) * C] * scale[:, l * C:(l + 1) * C] + shift[:, l * C:(l + 1) * C]
        h = jnp.where(h >= 0, h, 0.01 * h)
        acc = acc + jnp.dot(h, fcw_ref[...][(l + 1) * C:(l + 2) * C, :],
                            preferred_element_type=jnp.float32)
    out_ref[...] = acc


def _final(data, conv, scale, shift, fc_W):
    return pl.pallas_call(
        _final_body,
        grid=(G,),
        in_specs=[
            pl.BlockSpec((R, C), lambda i: (i, 0)),
            pl.BlockSpec((R, DW), lambda i: (i, 0)),
            pl.BlockSpec((8, DW), lambda i: (0, 0)),
            pl.BlockSpec((8, DW), lambda i: (0, 0)),
            pl.BlockSpec((C * (L + 1), C), lambda i: (0, 0)),
        ],
        out_specs=pl.BlockSpec((R, C), lambda i: (i, 0)),
        out_shape=jax.ShapeDtypeStruct((N, C), jnp.float32),
    )(data, conv, scale, shift, fc_W)


def kernel(data, edge_index, W_bases, W_comb, b_comb, conv_bias, bn_gamma, bn_beta, fc_W, fc_b):
    src, dst = edge_index[0], edge_index[1]
    Wb_all = jnp.concatenate([W_bases[l] for l in range(L)], axis=1)
    Wc_all = jnp.concatenate([W_comb[l] for l in range(L)], axis=1)
    bc_all = jnp.broadcast_to(jnp.concatenate([b_comb[l] for l in range(L)], axis=0)[None, :], (8, DW))

    bases, w_all = _proj(data, Wb_all, Wc_all, bc_all)

    deg_e = jax.ops.segment_sum(jnp.ones(E, jnp.float32), dst, N)
    deg = deg_e + 1.0
    dis = jax.lax.rsqrt(deg)
    bases2 = dis[:, None] * bases
    sum_e = jax.ops.segment_sum(bases[src], dst, N)
    max_e = jax.ops.segment_max(bases[src], dst, N)
    max_e = jnp.where(jnp.isfinite(max_e), max_e, -1e30)
    sym_e = jax.ops.segment_sum(bases2[src], dst, N)

    rdeg = (1.0 / deg)[:, None]
    conv, stats = _combine(bases, w_all, sum_e, max_e, sym_e, rdeg, dis[:, None])

    mu = stats[0] / N
    var = stats[1] / N - mu * mu
    g = jnp.concatenate([bn_gamma[l] for l in range(L)], axis=0)
    b = jnp.concatenate([bn_beta[l] for l in range(L)], axis=0)
    scale = g * jax.lax.rsqrt(var + 1e-5)
    shift = b - mu * scale
    scale8 = jnp.broadcast_to(scale[None, :], (8, DW))
    shift8 = jnp.broadcast_to(shift[None, :], (8, DW))

    out = _final(data, conv, scale8, shift8, fc_W)
    return out + fc_b


# speedup vs baseline: 3.3001x; 1.3411x over previous
import functools
import jax
import jax.numpy as jnp
from jax import lax
from jax.experimental import pallas as pl
from jax.experimental.pallas import tpu as pltpu
from jax.experimental.pallas import tpu_sc as plsc

N = 10000
E = 320000
C = 128
H = 8
NB = 4
A = 4
FH = C // H
L = 3
D = NB * FH * L
DW = H * NB * A * L
R = 400
G = N // R


NC, NS, LN = 2, 16, 16
NW = NC * NS
RT = 320
NP = NW * RT
SCR = NS * RT
TRASH = SCR
SROWS = SCR + 8
CH = 2000
NCH = E // CH
NV = CH // LN
B = 64
CAP = CH + B + LN

_sc_mesh = plsc.VectorSubcoreMesh(core_axis_name="c", subcore_axis_name="s")


@functools.partial(
    pl.kernel,
    out_type=jax.ShapeDtypeStruct((NW, 320), jnp.float32),
    mesh=_sc_mesh,
    scratch_types=[
        pltpu.VMEM((CH,), jnp.int32),
        pltpu.VMEM((320,), jnp.float32),
    ],
    compiler_params=pltpu.CompilerParams(needs_layout_passes=False),
)
def _sc_deg(dst_hbm, deg_hbm, dstch, degacc):
    sc = lax.axis_index("c")
    sub = lax.axis_index("s")
    wid = sc * NS + sub
    lo = wid * RT
    hi = lo + RT
    for k in range(320 // LN):
        degacc[pl.ds(k * LN, LN)] = jnp.zeros((LN,), jnp.float32)
    ones = jnp.ones((LN,), jnp.float32)

    def chunk(ci, _):
        pltpu.sync_copy(dst_hbm.at[pl.ds(ci * CH, CH)], dstch)

        def scan(v, _2):
            d16 = dstch[pl.ds(v * LN, LN)]
            m = (d16 >= lo) & (d16 < hi)
            plsc.addupdate_scatter(degacc, [d16 - lo], ones, mask=m)
            return 0

        lax.fori_loop(0, NV, scan, 0)
        return 0

    lax.fori_loop(0, NCH, chunk, 0)
    pltpu.sync_copy(degacc, deg_hbm.at[wid])


DH = D // 2


@functools.partial(
    pl.kernel,
    out_type=[jax.ShapeDtypeStruct((NP, DH), jnp.float32)] * 6,
    mesh=_sc_mesh,
    scratch_types=[
        pltpu.VMEM((CH,), jnp.int32),
        pltpu.VMEM((CH,), jnp.int32),
        pltpu.VMEM((CAP,), jnp.int32),
        pltpu.VMEM((CAP,), jnp.int32),
        pltpu.VMEM((8, B), jnp.int32),
        pltpu.VMEM((B, DH), jnp.float32),
        pltpu.VMEM((B, DH), jnp.float32),
        pltpu.VMEM((RT, DH), jnp.float32),
        pltpu.VMEM_SHARED((SROWS, DH), jnp.float32),
        pltpu.VMEM_SHARED((SROWS, DH), jnp.float32),
        pltpu.SemaphoreType.DMA,
        pltpu.SemaphoreType.DMA,
    ],
    compiler_params=pltpu.CompilerParams(needs_layout_passes=False, use_tc_tiling_on_sc=False),
)
def _sc_main(ba0, ba1, bb0, bb1, src_hbm, dst_hbm,
             sm0, sm1, mx0, mx1, sy0, sy1,
             srcch, dstch, csrc, cdl, idxrow, msg, msg2, accmax, ssum, ssym,
             sem, sem2):
    sc = lax.axis_index("c")
    sub = lax.axis_index("s")
    wid = sc * NS + sub
    lo = wid * RT
    hi = lo + RT
    scb = sc * SCR
    subb = sub * RT
    NKV = DH // LN

    for p, (tb, tb2, so, mo, yo) in enumerate(
            ((ba0, bb0, sm0, mx0, sy0), (ba1, bb1, sm1, mx1, sy1))):

        def initmax(r, _):
            for k in range(NKV):
                accmax[r, pl.ds(k * LN, LN)] = jnp.full((LN,), -1e30, jnp.float32)
            return 0

        lax.fori_loop(0, RT, initmax, 0)

        def zmsg(r, _):
            for k in range(NKV):
                msg[r, pl.ds(k * LN, LN)] = jnp.zeros((LN,), jnp.float32)
            return 0

        lax.fori_loop(0, B, zmsg, 0)
        for j in range(RT // B):
            pltpu.sync_copy(msg, ssum.at[pl.ds(subb + j * B, B)])
            pltpu.sync_copy(msg, ssym.at[pl.ds(subb + j * B, B)])

        @pl.when(sub == NS - 1)
        def _():
            pltpu.sync_copy(msg.at[pl.ds(0, SROWS - SCR)], ssum.at[pl.ds(SCR, SROWS - SCR)])
            pltpu.sync_copy(msg.at[pl.ds(0, SROWS - SCR)], ssym.at[pl.ds(SCR, SROWS - SCR)])

        plsc.subcore_barrier()

        def batch(off, nmax):
            idx = csrc.at[pl.ds(off, B)]
            pltpu.async_copy(tb.at[idx], msg, sem).wait()
            pltpu.async_copy(tb2.at[idx], msg2, sem2).wait()
            for k in range(B // LN):
                idxrow[0, pl.ds(k * LN, LN)] = cdl[pl.ds(off + k * LN, LN)]
            pltpu.sync_copy(msg, ssum.at[idxrow.at[0]], add=True)
            pltpu.sync_copy(msg2, ssym.at[idxrow.at[0]], add=True)

            def emax(e, _):
                row = cdl[pl.ds(off + e, LN)][0] - subb
                for k in range(NKV):
                    sl = pl.ds(k * LN, LN)
                    accmax[row, sl] = jnp.maximum(accmax[row, sl], msg[e, sl])
                return 0

            lax.fori_loop(0, nmax, emax, 0)

        def chunk(ci, cnt):
            pltpu.sync_copy(src_hbm.at[pl.ds(ci * CH, CH)], srcch)
            pltpu.sync_copy(dst_hbm.at[pl.ds(ci * CH, CH)], dstch)

            def scan(v, c):
                d16 = dstch[pl.ds(v * LN, LN)]
                s16 = srcch[pl.ds(v * LN, LN)]
                m = (d16 >= lo) & (d16 < hi)
                plsc.store_compressed(csrc.at[pl.ds(c, LN)], s16, mask=m)
                plsc.store_compressed(cdl.at[pl.ds(c, LN)], d16 - scb, mask=m)
                return c + plsc.all_reduce_population_count(m)[0]

            cnt = lax.fori_loop(0, NV, scan, cnt)
            nb = cnt // B

            def dob(b, _):
                batch(b * B, B)
                return 0

            lax.fori_loop(0, nb, dob, 0)
            rem = cnt - nb * B

            @pl.when(nb > 0)
            def _():
                off = nb * B
                for k in range(B // LN):
                    t = csrc[pl.ds(off + k * LN, LN)]
                    csrc[pl.ds(k * LN, LN)] = t
                    t2 = cdl[pl.ds(off + k * LN, LN)]
                    cdl[pl.ds(k * LN, LN)] = t2

            return rem

        rem = lax.fori_loop(0, NCH, chunk, 0)

        for k in range(B // LN):
            csrc[pl.ds(rem + k * LN, LN)] = jnp.zeros((LN,), jnp.int32)
            cdl[pl.ds(rem + k * LN, LN)] = jnp.full((LN,), TRASH, jnp.int32)

        @pl.when(rem > 0)
        def _():
            batch(0, rem)

        plsc.subcore_barrier()
        pltpu.sync_copy(accmax, mo.at[pl.ds(lo, RT)])
        pltpu.sync_copy(ssum.at[pl.ds(subb, RT)], so.at[pl.ds(lo, RT)])
        pltpu.sync_copy(ssym.at[pl.ds(subb, RT)], yo.at[pl.ds(lo, RT)])


def _scale_body(bases_ref, dis_ref, out_ref):
    out_ref[...] = bases_ref[...] * dis_ref[...]


def _scale(bases, dis):
    return pl.pallas_call(
        _scale_body,
        grid=(G,),
        in_specs=[
            pl.BlockSpec((R, D), lambda i: (i, 0)),
            pl.BlockSpec((R, 1), lambda i: (i, 0)),
        ],
        out_specs=pl.BlockSpec((R, D), lambda i: (i, 0)),
        out_shape=jax.ShapeDtypeStruct((N, D), jnp.float32),
    )(bases, dis)


def _proj_body(x_ref, wb_ref, wc_ref, bc_ref, bases_ref, w_ref):
    x = x_ref[...]
    bases_ref[...] = jnp.dot(x, wb_ref[...], preferred_element_type=jnp.float32)
    w_ref[...] = jnp.dot(x, wc_ref[...], preferred_element_type=jnp.float32) + bc_ref[...][0:1, :]


def _proj(data, Wb_all, Wc_all, bc_all):
    return pl.pallas_call(
        _proj_body,
        grid=(G,),
        in_specs=[
            pl.BlockSpec((R, C), lambda i: (i, 0)),
            pl.BlockSpec((C, D), lambda i: (0, 0)),
            pl.BlockSpec((C, DW), lambda i: (0, 0)),
            pl.BlockSpec((8, DW), lambda i: (0, 0)),
        ],
        out_specs=[
            pl.BlockSpec((R, D), lambda i: (i, 0)),
            pl.BlockSpec((R, DW), lambda i: (i, 0)),
        ],
        out_shape=[
            jax.ShapeDtypeStruct((N, D), jnp.float32),
            jax.ShapeDtypeStruct((N, DW), jnp.float32),
        ],
    )(data, Wb_all, Wc_all, bc_all)


def _combine_body(bases_ref, w_ref, sum_ref, max_ref, sym_ref, rdeg_ref, dis_ref,
                  conv_ref, stats_ref):
    step = pl.program_id(0)
    bases = bases_ref[...]
    rdeg = rdeg_ref[...]
    dis = dis_ref[...]
    dis2 = dis * dis
    outs = []
    for l in range(L):
        sl = slice(l * 64, (l + 1) * 64)
        b_l = bases[:, sl]
        s_full = sum_ref[...][:, sl] + b_l
        mean = s_full * rdeg
        mx = jnp.maximum(max_ref[...][:, sl], b_l)
        sym = dis * sym_ref[...][:, sl] + dis2 * b_l
        aggs = (mean, mx, s_full, sym)
        w3 = w_ref[...][:, l * 128:(l + 1) * 128].reshape(R, H, 16)
        acc = jnp.zeros((R, C), jnp.float32)
        for j in range(16):
            a_, b_ = j // NB, j % NB
            aj = aggs[a_][:, b_ * FH:(b_ + 1) * FH]
            ajt = jnp.concatenate([aj] * H, axis=1)
            wj = jnp.broadcast_to(w3[:, :, j:j + 1], (R, H, FH)).reshape(R, C)
            acc = acc + wj * ajt
        outs.append(acc)
    h_all = jnp.concatenate(outs, axis=1)
    conv_ref[...] = h_all
    psum = jnp.sum(h_all, axis=0, keepdims=True)
    psumsq = jnp.sum(h_all * h_all, axis=0, keepdims=True)
    blk = jnp.concatenate([psum, psumsq, jnp.zeros((6, DW), jnp.float32)], axis=0)

    @pl.when(step == 0)
    def _():
        stats_ref[...] = jnp.zeros((8, DW), jnp.float32)

    stats_ref[...] += blk


def _combine(bases, w_all, sum_e, max_e, sym_e, rdeg, dis):
    return pl.pallas_call(
        _combine_body,
        grid=(G,),
        in_specs=[
            pl.BlockSpec((R, D), lambda i: (i, 0)),
            pl.BlockSpec((R, DW), lambda i: (i, 0)),
            pl.BlockSpec((R, D), lambda i: (i, 0)),
            pl.BlockSpec((R, D), lambda i: (i, 0)),
            pl.BlockSpec((R, D), lambda i: (i, 0)),
            pl.BlockSpec((R, 1), lambda i: (i, 0)),
            pl.BlockSpec((R, 1), lambda i: (i, 0)),
        ],
        out_specs=[
            pl.BlockSpec((R, DW), lambda i: (i, 0)),
            pl.BlockSpec((8, DW), lambda i: (0, 0)),
        ],
        out_shape=[
            jax.ShapeDtypeStruct((N, DW), jnp.float32),
            jax.ShapeDtypeStruct((8, DW), jnp.float32),
        ],
    )(bases, w_all, sum_e, max_e, sym_e, rdeg, dis)


def _final_body(data_ref, conv_ref, scale_ref, shift_ref, fcw_ref, out_ref):
    acc = jnp.dot(data_ref[...], fcw_ref[...][0:C, :],
                  preferred_element_type=jnp.float32)
    conv = conv_ref[...]
    scale = scale_ref[...][0:1, :]
    shift = shift_ref[...][0:1, :]
    for l in range(L):
        h = conv[:, l * C:(l + 1) * C] * scale[:, l * C:(l + 1) * C] + shift[:, l * C:(l + 1) * C]
        h = jnp.where(h >= 0, h, 0.01 * h)
        acc = acc + jnp.dot(h, fcw_ref[...][(l + 1) * C:(l + 2) * C, :],
                            preferred_element_type=jnp.float32)
    out_ref[...] = acc


def _final(data, conv, scale, shift, fc_W):
    return pl.pallas_call(
        _final_body,
        grid=(G,),
        in_specs=[
            pl.BlockSpec((R, C), lambda i: (i, 0)),
            pl.BlockSpec((R, DW), lambda i: (i, 0)),
            pl.BlockSpec((8, DW), lambda i: (0, 0)),
            pl.BlockSpec((8, DW), lambda i: (0, 0)),
            pl.BlockSpec((C * (L + 1), C), lambda i: (0, 0)),
        ],
        out_specs=pl.BlockSpec((R, C), lambda i: (i, 0)),
        out_shape=jax.ShapeDtypeStruct((N, C), jnp.float32),
    )(data, conv, scale, shift, fc_W)


def kernel(data, edge_index, W_bases, W_comb, b_comb, conv_bias, bn_gamma, bn_beta, fc_W, fc_b):
    Wb_all = jnp.concatenate([W_bases[l] for l in range(L)], axis=1)
    Wc_all = jnp.concatenate([W_comb[l] for l in range(L)], axis=1)
    bc_all = jnp.broadcast_to(jnp.concatenate([b_comb[l] for l in range(L)], axis=0)[None, :], (8, DW))

    bases, w_all = _proj(data, Wb_all, Wc_all, bc_all)

    src = edge_index[0]
    dst = edge_index[1]
    deg2d = _sc_deg(dst)
    deg = deg2d[:, :RT].reshape(NP)[:N] + 1.0
    dis = jax.lax.rsqrt(deg)
    bases2 = _scale(bases, dis[:, None])
    sm0, sm1, mx0, mx1, sy0, sy1 = _sc_main(
        bases[:, :DH], bases[:, DH:], bases2[:, :DH], bases2[:, DH:], src, dst)
    sum_e = jnp.concatenate([sm0, sm1], axis=1)[:N]
    max_e = jnp.concatenate([mx0, mx1], axis=1)[:N]
    sym_e = jnp.concatenate([sy0, sy1], axis=1)[:N]

    rdeg = (1.0 / deg)[:, None]
    conv, stats = _combine(bases, w_all, sum_e, max_e, sym_e, rdeg, dis[:, None])

    mu = stats[0] / N
    var = stats[1] / N - mu * mu
    g = jnp.concatenate([bn_gamma[l] for l in range(L)], axis=0)
    b = jnp.concatenate([bn_beta[l] for l in range(L)], axis=0)
    scale = g * jax.lax.rsqrt(var + 1e-5)
    shift = b - mu * scale
    scale8 = jnp.broadcast_to(scale[None, :], (8, DW))
    shift8 = jnp.broadcast_to(shift[None, :], (8, DW))

    out = _final(data, conv, scale8, shift8, fc_W)
    return out + fc_b


# speedup vs baseline: 4.1063x; 1.2443x over previous
import functools
import jax
import jax.numpy as jnp
from jax import lax
from jax.experimental import pallas as pl
from jax.experimental.pallas import tpu as pltpu
from jax.experimental.pallas import tpu_sc as plsc

N = 10000
E = 320000
C = 128
H = 8
NB = 4
A = 4
FH = C // H
L = 3
D = NB * FH * L
DW = H * NB * A * L
R = 400
G = N // R


NC, NS, LN = 2, 16, 16
NW = NC * NS
RT = 320
NP = NW * RT
SCR = NS * RT
TRASH = SCR
SROWS = SCR + 8
CH = 2000
NCH = E // CH
NV = CH // LN
B = 64
CAP = CH + 2 * B + 2 * LN

_sc_mesh = plsc.VectorSubcoreMesh(core_axis_name="c", subcore_axis_name="s")


@functools.partial(
    pl.kernel,
    out_type=jax.ShapeDtypeStruct((NW, 320), jnp.float32),
    mesh=_sc_mesh,
    scratch_types=[
        pltpu.VMEM((CH,), jnp.int32),
        pltpu.VMEM((320,), jnp.float32),
    ],
    compiler_params=pltpu.CompilerParams(needs_layout_passes=False),
)
def _sc_deg(dst_hbm, deg_hbm, dstch, degacc):
    sc = lax.axis_index("c")
    sub = lax.axis_index("s")
    wid = sc * NS + sub
    lo = wid * RT
    hi = lo + RT
    for k in range(320 // LN):
        degacc[pl.ds(k * LN, LN)] = jnp.zeros((LN,), jnp.float32)
    ones = jnp.ones((LN,), jnp.float32)

    def chunk(ci, _):
        pltpu.sync_copy(dst_hbm.at[pl.ds(ci * CH, CH)], dstch)

        def scan(v, _2):
            d16 = dstch[pl.ds(v * LN, LN)]
            m = (d16 >= lo) & (d16 < hi)
            plsc.addupdate_scatter(degacc, [d16 - lo], ones, mask=m)
            return 0

        lax.fori_loop(0, NV, scan, 0)
        return 0

    lax.fori_loop(0, NCH, chunk, 0)
    pltpu.sync_copy(degacc, deg_hbm.at[wid])


DH = D // 2
B2 = 2 * B


@functools.partial(
    pl.kernel,
    out_type=[jax.ShapeDtypeStruct((NP, DH), jnp.float32)] * 6,
    mesh=_sc_mesh,
    scratch_types=[
        pltpu.VMEM((CH,), jnp.int32),
        pltpu.VMEM((CH,), jnp.int32),
        pltpu.VMEM((CH,), jnp.int32),
        pltpu.VMEM((CH,), jnp.int32),
        pltpu.VMEM((CAP,), jnp.int32),
        pltpu.VMEM((CAP,), jnp.int32),
        pltpu.VMEM((8, B), jnp.int32),
        pltpu.VMEM((8, B), jnp.int32),
        pltpu.VMEM((B, DH), jnp.float32),
        pltpu.VMEM((B, DH), jnp.float32),
        pltpu.VMEM((B, DH), jnp.float32),
        pltpu.VMEM((B, DH), jnp.float32),
        pltpu.VMEM((RT, DH), jnp.float32),
        pltpu.VMEM_SHARED((SROWS, DH), jnp.float32),
        pltpu.VMEM_SHARED((SROWS, DH), jnp.float32),
    ] + [pltpu.SemaphoreType.DMA] * 12,
    compiler_params=pltpu.CompilerParams(needs_layout_passes=False, use_tc_tiling_on_sc=False),
)
def _sc_main(ba0, ba1, bb0, bb1, src_hbm, dst_hbm,
             sm0, sm1, mx0, mx1, sy0, sy1,
             srcch0, dstch0, srcch1, dstch1, csrc, cdl, idxrow0, idxrow1,
             msgA, msgA2, msgB, msgB2, accmax, ssum, ssym,
             sga0, sga1, sgb0, sgb1, ssa0, ssa1, ssb0, ssb1,
             sst0, sst1, sst2, sst3):
    sc = lax.axis_index("c")
    sub = lax.axis_index("s")
    wid = sc * NS + sub
    lo = wid * RT
    hi = lo + RT
    scb = sc * SCR
    subb = sub * RT
    NKV = DH // LN

    def emax(buf, base, nmax):
        def body(e, _):
            row = cdl[pl.ds(base + e, LN)][0] - subb
            for k in range(NKV):
                sl = pl.ds(k * LN, LN)
                accmax[row, sl] = jnp.maximum(accmax[row, sl], buf[e, sl])
            return 0

        lax.fori_loop(0, nmax, body, 0)

    def scan_chunk(srcb, dstb, cnt):
        def scan(v, c):
            d16 = dstb[pl.ds(v * LN, LN)]
            s16 = srcb[pl.ds(v * LN, LN)]
            m = (d16 >= lo) & (d16 < hi)
            plsc.store_compressed(csrc.at[pl.ds(c, LN)], s16, mask=m)
            plsc.store_compressed(cdl.at[pl.ds(c, LN)], d16 - scb, mask=m)
            return c + plsc.all_reduce_population_count(m)[0]

        return lax.fori_loop(0, NV, scan, cnt)

    for p, (tb, tb2, so, mo, yo) in enumerate(
            ((ba0, bb0, sm0, mx0, sy0), (ba1, bb1, sm1, mx1, sy1))):

        def initmax(r, _):
            for k in range(NKV):
                accmax[r, pl.ds(k * LN, LN)] = jnp.full((LN,), -1e30, jnp.float32)
            return 0

        lax.fori_loop(0, RT, initmax, 0)

        def zmsg(r, _):
            for k in range(NKV):
                msgA[r, pl.ds(k * LN, LN)] = jnp.zeros((LN,), jnp.float32)
            return 0

        lax.fori_loop(0, B, zmsg, 0)
        for j in range(RT // B):
            pltpu.sync_copy(msgA, ssum.at[pl.ds(subb + j * B, B)])
            pltpu.sync_copy(msgA, ssym.at[pl.ds(subb + j * B, B)])
        tailr = RT - (RT // B) * B
        if tailr:
            pltpu.sync_copy(msgA.at[pl.ds(0, tailr)], ssum.at[pl.ds(subb + RT - tailr, tailr)])
            pltpu.sync_copy(msgA.at[pl.ds(0, tailr)], ssym.at[pl.ds(subb + RT - tailr, tailr)])

        @pl.when(sub == NS - 1)
        def _():
            pltpu.sync_copy(msgA.at[pl.ds(0, SROWS - SCR)], ssum.at[pl.ds(SCR, SROWS - SCR)])
            pltpu.sync_copy(msgA.at[pl.ds(0, SROWS - SCR)], ssym.at[pl.ds(SCR, SROWS - SCR)])

        plsc.subcore_barrier()

        def wcond(st):
            return st[1] - st[0] >= B2

        def wbody(st):
            off, cnt = st
            off = pl.multiple_of(off, B)
            idx0 = csrc.at[pl.ds(off, B)]
            idx1 = csrc.at[pl.ds(off + B, B)]
            g0 = pltpu.async_copy(tb.at[idx0], msgA, sga0)
            g0b = pltpu.async_copy(tb2.at[idx0], msgA2, sga1)
            g1 = pltpu.async_copy(tb.at[idx1], msgB, sgb0)
            g1b = pltpu.async_copy(tb2.at[idx1], msgB2, sgb1)
            g0.wait()
            g0b.wait()
            for k in range(B // LN):
                idxrow0[0, pl.ds(k * LN, LN)] = cdl[pl.ds(off + k * LN, LN)]
            s0 = pltpu.async_copy(msgA, ssum.at[idxrow0.at[0]], ssa0, add=True)
            s0b = pltpu.async_copy(msgA2, ssym.at[idxrow0.at[0]], ssa1, add=True)
            emax(msgA, off, B)
            g1.wait()
            g1b.wait()
            for k in range(B // LN):
                idxrow1[0, pl.ds(k * LN, LN)] = cdl[pl.ds(off + B + k * LN, LN)]
            s1 = pltpu.async_copy(msgB, ssum.at[idxrow1.at[0]], ssb0, add=True)
            s1b = pltpu.async_copy(msgB2, ssym.at[idxrow1.at[0]], ssb1, add=True)
            emax(msgB, off + B, B)
            s0.wait()
            s0b.wait()
            s1.wait()
            s1b.wait()
            return (off + B2, cnt)

        def drain_and_slide(cnt):
            off, cnt = lax.while_loop(wcond, wbody, (0, cnt))

            @pl.when(off > 0)
            def _():
                for k in range(B2 // LN):
                    t = csrc[pl.ds(off + k * LN, LN)]
                    csrc[pl.ds(k * LN, LN)] = t
                    t2 = cdl[pl.ds(off + k * LN, LN)]
                    cdl[pl.ds(k * LN, LN)] = t2

            return cnt - off

        pltpu.async_copy(src_hbm.at[pl.ds(0, CH)], srcch0, sst0)
        pltpu.async_copy(dst_hbm.at[pl.ds(0, CH)], dstch0, sst1)

        def cpair(ci2, cnt):
            ci = ci2 * 2
            pltpu.async_copy(src_hbm.at[pl.ds((ci + 1) * CH, CH)], srcch1, sst2)
            pltpu.async_copy(dst_hbm.at[pl.ds((ci + 1) * CH, CH)], dstch1, sst3)
            pltpu.make_async_copy(src_hbm.at[pl.ds(0, CH)], srcch0, sst0).wait()
            pltpu.make_async_copy(dst_hbm.at[pl.ds(0, CH)], dstch0, sst1).wait()
            cnt = scan_chunk(srcch0, dstch0, cnt)
            cnt = drain_and_slide(cnt)

            @pl.when(ci2 < NCH // 2 - 1)
            def _():
                pltpu.async_copy(src_hbm.at[pl.ds((ci + 2) * CH, CH)], srcch0, sst0)
                pltpu.async_copy(dst_hbm.at[pl.ds((ci + 2) * CH, CH)], dstch0, sst1)

            pltpu.make_async_copy(src_hbm.at[pl.ds(0, CH)], srcch1, sst2).wait()
            pltpu.make_async_copy(dst_hbm.at[pl.ds(0, CH)], dstch1, sst3).wait()
            cnt = scan_chunk(srcch1, dstch1, cnt)
            cnt = drain_and_slide(cnt)
            return cnt

        cnt = lax.fori_loop(0, NCH // 2, cpair, 0)

        for k in range(B // LN):
            csrc[pl.ds(cnt + k * LN, LN)] = jnp.zeros((LN,), jnp.int32)
            cdl[pl.ds(cnt + k * LN, LN)] = jnp.full((LN,), TRASH, jnp.int32)

        nbf = (cnt + B - 1) // B

        def fb(b, _):
            off = b * B
            idx = csrc.at[pl.ds(off, B)]
            pltpu.async_copy(tb.at[idx], msgA, sga0).wait()
            pltpu.async_copy(tb2.at[idx], msgA2, sga1).wait()
            for k in range(B // LN):
                idxrow0[0, pl.ds(k * LN, LN)] = cdl[pl.ds(off + k * LN, LN)]
            pltpu.sync_copy(msgA, ssum.at[idxrow0.at[0]], add=True)
            pltpu.sync_copy(msgA2, ssym.at[idxrow0.at[0]], add=True)
            emax(msgA, off, jnp.minimum(cnt - off, B))
            return 0

        lax.fori_loop(0, nbf, fb, 0)

        plsc.subcore_barrier()
        pltpu.sync_copy(accmax, mo.at[pl.ds(lo, RT)])
        pltpu.sync_copy(ssum.at[pl.ds(subb, RT)], so.at[pl.ds(lo, RT)])
        pltpu.sync_copy(ssym.at[pl.ds(subb, RT)], yo.at[pl.ds(lo, RT)])


def _scale_body(bases_ref, dis_ref, out_ref):
    out_ref[...] = bases_ref[...] * dis_ref[...]


def _scale(bases, dis):
    return pl.pallas_call(
        _scale_body,
        grid=(G,),
        in_specs=[
            pl.BlockSpec((R, D), lambda i: (i, 0)),
            pl.BlockSpec((R, 1), lambda i: (i, 0)),
        ],
        out_specs=pl.BlockSpec((R, D), lambda i: (i, 0)),
        out_shape=jax.ShapeDtypeStruct((N, D), jnp.float32),
    )(bases, dis)


def _proj_body(x_ref, wb_ref, wc_ref, bc_ref, bases_ref, w_ref):
    x = x_ref[...]
    bases_ref[...] = jnp.dot(x, wb_ref[...], preferred_element_type=jnp.float32)
    w_ref[...] = jnp.dot(x, wc_ref[...], preferred_element_type=jnp.float32) + bc_ref[...][0:1, :]


def _proj(data, Wb_all, Wc_all, bc_all):
    return pl.pallas_call(
        _proj_body,
        grid=(G,),
        in_specs=[
            pl.BlockSpec((R, C), lambda i: (i, 0)),
            pl.BlockSpec((C, D), lambda i: (0, 0)),
            pl.BlockSpec((C, DW), lambda i: (0, 0)),
            pl.BlockSpec((8, DW), lambda i: (0, 0)),
        ],
        out_specs=[
            pl.BlockSpec((R, D), lambda i: (i, 0)),
            pl.BlockSpec((R, DW), lambda i: (i, 0)),
        ],
        out_shape=[
            jax.ShapeDtypeStruct((N, D), jnp.float32),
            jax.ShapeDtypeStruct((N, DW), jnp.float32),
        ],
    )(data, Wb_all, Wc_all, bc_all)


def _combine_body(bases_ref, w_ref, sum_ref, max_ref, sym_ref, rdeg_ref, dis_ref,
                  conv_ref, stats_ref):
    step = pl.program_id(0)
    bases = bases_ref[...]
    rdeg = rdeg_ref[...]
    dis = dis_ref[...]
    dis2 = dis * dis
    outs = []
    for l in range(L):
        sl = slice(l * 64, (l + 1) * 64)
        b_l = bases[:, sl]
        s_full = sum_ref[...][:, sl] + b_l
        mean = s_full * rdeg
        mx = jnp.maximum(max_ref[...][:, sl], b_l)
        sym = dis * sym_ref[...][:, sl] + dis2 * b_l
        aggs = (mean, mx, s_full, sym)
        w3 = w_ref[...][:, l * 128:(l + 1) * 128].reshape(R, H, 16)
        acc = jnp.zeros((R, C), jnp.float32)
        for j in range(16):
            a_, b_ = j // NB, j % NB
            aj = aggs[a_][:, b_ * FH:(b_ + 1) * FH]
            ajt = jnp.concatenate([aj] * H, axis=1)
            wj = jnp.broadcast_to(w3[:, :, j:j + 1], (R, H, FH)).reshape(R, C)
            acc = acc + wj * ajt
        outs.append(acc)
    h_all = jnp.concatenate(outs, axis=1)
    conv_ref[...] = h_all
    psum = jnp.sum(h_all, axis=0, keepdims=True)
    psumsq = jnp.sum(h_all * h_all, axis=0, keepdims=True)
    blk = jnp.concatenate([psum, psumsq, jnp.zeros((6, DW), jnp.float32)], axis=0)

    @pl.when(step == 0)
    def _():
        stats_ref[...] = jnp.zeros((8, DW), jnp.float32)

    stats_ref[...] += blk


def _combine(bases, w_all, sum_e, max_e, sym_e, rdeg, dis):
    return pl.pallas_call(
        _combine_body,
        grid=(G,),
        in_specs=[
            pl.BlockSpec((R, D), lambda i: (i, 0)),
            pl.BlockSpec((R, DW), lambda i: (i, 0)),
            pl.BlockSpec((R, D), lambda i: (i, 0)),
            pl.BlockSpec((R, D), lambda i: (i, 0)),
            pl.BlockSpec((R, D), lambda i: (i, 0)),
            pl.BlockSpec((R, 1), lambda i: (i, 0)),
            pl.BlockSpec((R, 1), lambda i: (i, 0)),
        ],
        out_specs=[
            pl.BlockSpec((R, DW), lambda i: (i, 0)),
            pl.BlockSpec((8, DW), lambda i: (0, 0)),
        ],
        out_shape=[
            jax.ShapeDtypeStruct((N, DW), jnp.float32),
            jax.ShapeDtypeStruct((8, DW), jnp.float32),
        ],
    )(bases, w_all, sum_e, max_e, sym_e, rdeg, dis)


def _final_body(data_ref, conv_ref, scale_ref, shift_ref, fcw_ref, out_ref):
    acc = jnp.dot(data_ref[...], fcw_ref[...][0:C, :],
                  preferred_element_type=jnp.float32)
    conv = conv_ref[...]
    scale = scale_ref[...][0:1, :]
    shift = shift_ref[...][0:1, :]
    for l in range(L):
        h = conv[:, l * C:(l + 1) * C] * scale[:, l * C:(l + 1) * C] + shift[:, l * C:(l + 1) * C]
        h = jnp.where(h >= 0, h, 0.01 * h)
        acc = acc + jnp.dot(h, fcw_ref[...][(l + 1) * C:(l + 2) * C, :],
                            preferred_element_type=jnp.float32)
    out_ref[...] = acc


def _final(data, conv, scale, shift, fc_W):
    return pl.pallas_call(
        _final_body,
        grid=(G,),
        in_specs=[
            pl.BlockSpec((R, C), lambda i: (i, 0)),
            pl.BlockSpec((R, DW), lambda i: (i, 0)),
            pl.BlockSpec((8, DW), lambda i: (0, 0)),
            pl.BlockSpec((8, DW), lambda i: (0, 0)),
            pl.BlockSpec((C * (L + 1), C), lambda i: (0, 0)),
        ],
        out_specs=pl.BlockSpec((R, C), lambda i: (i, 0)),
        out_shape=jax.ShapeDtypeStruct((N, C), jnp.float32),
    )(data, conv, scale, shift, fc_W)


def kernel(data, edge_index, W_bases, W_comb, b_comb, conv_bias, bn_gamma, bn_beta, fc_W, fc_b):
    Wb_all = jnp.concatenate([W_bases[l] for l in range(L)], axis=1)
    Wc_all = jnp.concatenate([W_comb[l] for l in range(L)], axis=1)
    bc_all = jnp.broadcast_to(jnp.concatenate([b_comb[l] for l in range(L)], axis=0)[None, :], (8, DW))

    bases, w_all = _proj(data, Wb_all, Wc_all, bc_all)

    src = edge_index[0]
    dst = edge_index[1]
    deg2d = _sc_deg(dst)
    deg = deg2d[:, :RT].reshape(NP)[:N] + 1.0
    dis = jax.lax.rsqrt(deg)
    bases2 = _scale(bases, dis[:, None])
    sm0, sm1, mx0, mx1, sy0, sy1 = _sc_main(
        bases[:, :DH], bases[:, DH:], bases2[:, :DH], bases2[:, DH:], src, dst)
    sum_e = jnp.concatenate([sm0, sm1], axis=1)[:N]
    max_e = jnp.concatenate([mx0, mx1], axis=1)[:N]
    sym_e = jnp.concatenate([sy0, sy1], axis=1)[:N]

    rdeg = (1.0 / deg)[:, None]
    conv, stats = _combine(bases, w_all, sum_e, max_e, sym_e, rdeg, dis[:, None])

    mu = stats[0] / N
    var = stats[1] / N - mu * mu
    g = jnp.concatenate([bn_gamma[l] for l in range(L)], axis=0)
    b = jnp.concatenate([bn_beta[l] for l in range(L)], axis=0)
    scale = g * jax.lax.rsqrt(var + 1e-5)
    shift = b - mu * scale
    scale8 = jnp.broadcast_to(scale[None, :], (8, DW))
    shift8 = jnp.broadcast_to(shift[None, :], (8, DW))

    out = _final(data, conv, scale8, shift8, fc_W)
    return out + fc_b
